# Initial kernel scaffold; baseline (speedup 1.0000x reference)
import jax
import jax.numpy as jnp
from jax.experimental import pallas as pl


def kernel(x_in, table):
    raise NotImplementedError("write your pallas kernel here")


# speedup vs baseline: 1.2189x; 1.2189x over previous
import functools

import jax
import jax.numpy as jnp
import numpy as np
from jax import lax
from jax.experimental import pallas as pl
from jax.experimental.pallas import tpu as pltpu
from jax.experimental.pallas import tpu_sc as plsc

NOTES_POOL_SIZE = 1000000
MAX_POS = 2048
EMBED_DIM = 32
B = 4096
L = 200

_NC = 2
_NS = 16
_NW = _NC * _NS
_ROWS = B * L
_ROWS_PER_W = _ROWS // _NW
_CHUNK = 4 * L
_NCHUNK = _ROWS_PER_W // _CHUNK


def _positional_encoding_np(max_pos, embed_dim):
    pos = np.arange(max_pos)[:, np.newaxis]
    i = np.arange(embed_dim)[np.newaxis, :]
    angle_rates = 1.0 / np.power(10000, 2 * (i // 2) / np.float32(embed_dim))
    angle_rads = pos * angle_rates
    angle_rads[:, 0::2] = np.sin(angle_rads[:, 0::2])
    angle_rads[:, 1::2] = np.cos(angle_rads[:, 1::2])
    return angle_rads.astype(np.float32)


_POS_ENC = _positional_encoding_np(L, EMBED_DIM)


def _sc_embed(table, notes, dur, pos):
    mesh = plsc.VectorSubcoreMesh(core_axis_name="c", subcore_axis_name="s")

    @functools.partial(
        pl.kernel,
        mesh=mesh,
        compiler_params=pltpu.CompilerParams(
            use_tc_tiling_on_sc=False, needs_layout_passes=False),
        out_type=jax.ShapeDtypeStruct((_ROWS, EMBED_DIM), jnp.float32),
        scratch_types=[
            pltpu.VMEM((_CHUNK,), jnp.int32),
            pltpu.VMEM((_CHUNK,), jnp.float32),
            pltpu.VMEM((_CHUNK, EMBED_DIM), jnp.float32),
            pltpu.VMEM((L, EMBED_DIM), jnp.float32),
            pltpu.SemaphoreType.DMA,
        ],
    )
    def k(table_hbm, notes_hbm, dur_hbm, pos_hbm, out_hbm,
          idx_v, dur_v, rows_v, pos_v, sem):
        wid = lax.axis_index("s") * _NC + lax.axis_index("c")
        pltpu.sync_copy(pos_hbm, pos_v)

        def chunk_body(c, _):
            base = wid * _ROWS_PER_W + c * _CHUNK
            pltpu.sync_copy(notes_hbm.at[pl.ds(base, _CHUNK)], idx_v)
            pltpu.sync_copy(dur_hbm.at[pl.ds(base, _CHUNK)], dur_v)
            pltpu.async_copy(table_hbm.at[idx_v], rows_v, sem).wait()

            def row_body(r, _):
                lpos = lax.rem(r, L)
                dsplat = plsc.load_gather(
                    dur_v, [jnp.full((16,), r, jnp.int32)])
                h0 = rows_v[r, pl.ds(0, 16)] + pos_v[lpos, pl.ds(0, 16)] + dsplat
                h1 = rows_v[r, pl.ds(16, 16)] + pos_v[lpos, pl.ds(16, 16)] + dsplat
                rows_v[r, pl.ds(0, 16)] = h0
                rows_v[r, pl.ds(16, 16)] = h1
                return _

            lax.fori_loop(0, _CHUNK, row_body, 0, unroll=2)
            pltpu.sync_copy(rows_v, out_hbm.at[pl.ds(base, _CHUNK)])
            return _

        lax.fori_loop(0, _NCHUNK, chunk_body, 0)

    return k(table, notes, dur, pos)


@jax.jit
def kernel(x_in, table):
    notes = x_in[:, :, 0].reshape(-1)
    dur = x_in[:, :, 1].reshape(-1).astype(jnp.float32)
    pos = jnp.asarray(_POS_ENC)
    out = _sc_embed(table, notes, dur, pos)
    return out.reshape(B, L, EMBED_DIM)
